# baseline (speedup 1.0000x reference)
import jax
import jax.numpy as jnp
from jax import lax
from jax.experimental import pallas as pl
from jax.experimental.pallas import tpu as pltpu
from jax.experimental.pallas import tpu_sc as plsc

N_ATOMS_P = 100000
N_PAIRS_P = 3200000
N_FRAMES = 1024
NW = 32
PAIRS_PER_W = N_PAIRS_P // NW
CHUNK = 800
N_CHUNKS = PAIRS_PER_W // CHUNK
GROUPS = CHUNK // 16


def _lj_energy(sx, sy, sz, se, ss, dx_, dy_, dz_, de, ds_):
    ddx = sx - dx_
    ddy = sy - dy_
    ddz = sz - dz_
    r2 = ddx * ddx + ddy * ddy + ddz * ddz + 1e-12
    m = jnp.maximum(r2, 0.25)
    t = se * de + 1e-12
    i = plsc.bitcast(t, jnp.int32)
    i = jnp.int32(0x5F3759DF) - lax.shift_right_logical(i, 1)
    y = plsc.bitcast(i, jnp.float32)
    y = y * (1.5 - 0.5 * t * y * y)
    y = y * (1.5 - 0.5 * t * y * y)
    y = y * (1.5 - 0.5 * t * y * y)
    eps = t * y
    sig = 0.5 * (ss + ds_)
    sig2 = sig * sig
    sig6 = sig2 * sig2 * sig2
    m3 = m * m * m
    sr6 = sig6 / m3
    return 4.0 * eps * (sr6 * sr6 - sr6)


def _sc_body(table_hbm, src_hbm, dst_hbm, frames_hbm, out_hbm,
             idx_src0, idx_src1, idx_dst0, idx_dst1, frames_v0, frames_v1,
             rows_src0, rows_src1, rows_dst0, rows_dst1,
             acc, acc_red, idx64, shared_acc, table_sh, sem, sem2,
             sem_i0, sem_i1, sem_f0, sem_f1):
    cid = lax.axis_index("c")
    sid = lax.axis_index("s")
    wid = sid * 2 + cid
    idx_src = (idx_src0, idx_src1)
    idx_dst = (idx_dst0, idx_dst1)
    frames_v = (frames_v0, frames_v1)
    rows_src = (rows_src0, rows_src1)
    rows_dst = (rows_dst0, rows_dst1)
    sems = (sem, sem2)
    sems_i = (sem_i0, sem_i1)
    sems_f = (sem_f0, sem_f1)
    pair_base = wid * PAIRS_PER_W

    rows_per_tile = N_ATOMS_P // 16
    pltpu.sync_copy(table_hbm.at[pl.ds(sid * rows_per_tile, rows_per_tile)],
                    table_sh.at[pl.ds(sid * rows_per_tile, rows_per_tile)])

    zero16 = jnp.zeros((16,), jnp.float32)
    lane = lax.iota(jnp.int32, 16)

    def _zero(j, _):
        acc_red[j] = zero16
        for r in range(16):
            acc[r, pl.ds(j * 16, 16)] = zero16
        return 0
    lax.fori_loop(0, 64, _zero, 0)
    for j in range(4):
        idx64[pl.ds(j * 16, 16)] = lane + j * 16

    @pl.when(sid == 0)
    def _():
        pltpu.sync_copy(acc_red, shared_acc)
    plsc.subcore_barrier()

    f0 = jnp.zeros((16,), jnp.int32)
    f1 = f0 + 1
    f2 = f0 + 2
    f3 = f0 + 3
    f4 = f0 + 4

    def load_idx(c, b):
        pb = pair_base + c * CHUNK
        pltpu.async_copy(src_hbm.at[pl.ds(pb, CHUNK)], idx_src[b], sems_i[b])
        pltpu.async_copy(dst_hbm.at[pl.ds(pb, CHUNK)], idx_dst[b], sems_i[b])

    def wait_idx(c, b):
        pb = pair_base + c * CHUNK
        pltpu.make_async_copy(src_hbm.at[pl.ds(pb, CHUNK)], idx_src[b],
                              sems_i[b]).wait()
        pltpu.make_async_copy(dst_hbm.at[pl.ds(pb, CHUNK)], idx_dst[b],
                              sems_i[b]).wait()

    def load_frames(c, b):
        pb = pair_base + c * CHUNK
        pltpu.async_copy(frames_hbm.at[pl.ds(pb, CHUNK)], frames_v[b],
                         sems_f[b])

    def wait_frames(c, b):
        pb = pair_base + c * CHUNK
        pltpu.make_async_copy(frames_hbm.at[pl.ds(pb, CHUNK)], frames_v[b],
                              sems_f[b]).wait()

    def issue_gathers(b):
        pltpu.async_copy(table_sh.at[idx_src[b]], rows_src[b], sems[b])
        pltpu.async_copy(table_sh.at[idx_dst[b]], rows_dst[b], sems[b])

    def wait_gathers(b):
        pltpu.make_async_copy(table_sh.at[idx_src[b]], rows_src[b],
                              sems[b]).wait()
        pltpu.make_async_copy(table_sh.at[idx_dst[b]], rows_dst[b],
                              sems[b]).wait()

    def compute(b):
        rs, rd, fv = rows_src[b], rows_dst[b], frames_v[b]

        def group_body(g, _):
            r = g * 16 + lane
            fr = fv[pl.ds(g * 16, 16)]
            sx = plsc.load_gather(rs, [r, f0])
            sy = plsc.load_gather(rs, [r, f1])
            sz = plsc.load_gather(rs, [r, f2])
            se = plsc.load_gather(rs, [r, f3])
            ss = plsc.load_gather(rs, [r, f4])
            dx_ = plsc.load_gather(rd, [r, f0])
            dy_ = plsc.load_gather(rd, [r, f1])
            dz_ = plsc.load_gather(rd, [r, f2])
            de = plsc.load_gather(rd, [r, f3])
            ds_ = plsc.load_gather(rd, [r, f4])
            e = _lj_energy(sx, sy, sz, se, ss, dx_, dy_, dz_, de, ds_)
            plsc.addupdate_scatter(acc, [lane, fr], e)
            return 0
        lax.fori_loop(0, GROUPS, group_body, 0)

    load_idx(0, 0)
    load_frames(0, 0)
    wait_idx(0, 0)
    issue_gathers(0)
    load_idx(1, 1)
    load_frames(1, 1)
    wait_idx(1, 1)
    issue_gathers(1)

    def pipe_body(t, _):
        c0 = t * 2
        wait_gathers(0)
        load_idx(c0 + 2, 0)
        wait_frames(c0, 0)
        compute(0)
        load_frames(c0 + 2, 0)
        wait_idx(c0 + 2, 0)
        issue_gathers(0)
        wait_gathers(1)
        load_idx(c0 + 3, 1)
        wait_frames(c0 + 1, 1)
        compute(1)
        load_frames(c0 + 3, 1)
        wait_idx(c0 + 3, 1)
        issue_gathers(1)
        return 0
    lax.fori_loop(0, (N_CHUNKS - 3) // 2, pipe_body, 0)

    cL = N_CHUNKS - 1
    wait_gathers(0)
    load_idx(cL, 0)
    wait_frames(cL - 2, 0)
    compute(0)
    load_frames(cL, 0)
    wait_idx(cL, 0)
    issue_gathers(0)
    wait_gathers(1)
    wait_frames(cL - 1, 1)
    compute(1)
    wait_gathers(0)
    wait_frames(cL, 0)
    compute(0)

    def _red(j, _):
        s = acc[0, pl.ds(j * 16, 16)]
        for r in range(1, 16):
            s = s + acc[r, pl.ds(j * 16, 16)]
        acc_red[j] = s
        return 0
    lax.fori_loop(0, 64, _red, 0)

    pltpu.sync_copy(acc_red, shared_acc.at[idx64], add=True)
    plsc.subcore_barrier()

    @pl.when(sid == 0)
    def _():
        pltpu.sync_copy(shared_acc, out_hbm.at[cid])


@jax.jit
def _run(table, src1d, dst1d, frames1d):
    mesh = plsc.VectorSubcoreMesh(core_axis_name="c", subcore_axis_name="s")
    kern = pl.kernel(
        _sc_body,
        out_type=jax.ShapeDtypeStruct((2, 64, 16), jnp.float32),
        mesh=mesh,
        scratch_types=[
            pltpu.VMEM((CHUNK,), jnp.int32),
            pltpu.VMEM((CHUNK,), jnp.int32),
            pltpu.VMEM((CHUNK,), jnp.int32),
            pltpu.VMEM((CHUNK,), jnp.int32),
            pltpu.VMEM((CHUNK,), jnp.int32),
            pltpu.VMEM((CHUNK,), jnp.int32),
            pltpu.VMEM((CHUNK, 8), jnp.float32),
            pltpu.VMEM((CHUNK, 8), jnp.float32),
            pltpu.VMEM((CHUNK, 8), jnp.float32),
            pltpu.VMEM((CHUNK, 8), jnp.float32),
            pltpu.VMEM((16, N_FRAMES), jnp.float32),
            pltpu.VMEM((64, 16), jnp.float32),
            pltpu.VMEM((64,), jnp.int32),
            pltpu.VMEM_SHARED((64, 16), jnp.float32),
            pltpu.VMEM_SHARED((N_ATOMS_P, 8), jnp.float32),
            pltpu.SemaphoreType.DMA,
            pltpu.SemaphoreType.DMA,
            pltpu.SemaphoreType.DMA,
            pltpu.SemaphoreType.DMA,
            pltpu.SemaphoreType.DMA,
            pltpu.SemaphoreType.DMA,
        ],
        compiler_params=pltpu.CompilerParams(
            needs_layout_passes=False, use_tc_tiling_on_sc=False),
    )
    return kern(table, src1d, dst1d, frames1d)


def kernel(positions, atom_params, pair_index, frames, batch_size):
    n = positions.shape[0]
    table = jnp.concatenate(
        [positions, atom_params, jnp.zeros((n, 3), jnp.float32)], axis=1)
    out = _run(table, pair_index[0], pair_index[1],
               frames.astype(jnp.int32))
    return out.reshape(2, N_FRAMES).sum(axis=0)
